# baseline (speedup 1.0000x reference)
import functools

import jax
import jax.numpy as jnp
from jax import lax
from jax.experimental import pallas as pl
from jax.experimental.pallas import tpu as pltpu
from jax.experimental.pallas import tpu_sc as plsc

_VOCAB = 100001
_NUM_TRAINABLE = 1001
_CHAR_VOCAB = 1301
_WORD_DIM = 300
_CHAR_DIM = 200
_K = 5
_B, _S, _C = 1024, 20, 16
_WORD_RANGE = _VOCAB - _NUM_TRAINABLE
_D = _WORD_DIM + _CHAR_DIM
_BS = _B * _S

_NW = 32
_PER_TILE = _BS // _NW
_CHUNK = 64
_NCHUNK = _PER_TILE // _CHUNK
_NBUF = 3
_WPAD = 304

_R = 256
_NBLK = _BS // _R
_CV_PAD = 1312


def _sc_gather(words_flat, word_table, trainable_table):
    mesh = plsc.VectorSubcoreMesh(core_axis_name="c", subcore_axis_name="s")

    @functools.partial(
        pl.kernel,
        out_type=(
            jax.ShapeDtypeStruct((_BS, _WPAD), jnp.float32),
            jax.ShapeDtypeStruct((_BS, _WPAD), jnp.float32),
        ),
        mesh=mesh,
        compiler_params=pltpu.CompilerParams(use_tc_tiling_on_sc=False),
        scratch_types=(
            [pltpu.VMEM((_CHUNK,), jnp.int32)] * (2 * _NBUF)
            + [pltpu.VMEM((_CHUNK, _WPAD), jnp.float32)] * (2 * _NBUF)
            + [pltpu.SemaphoreType.DMA] * (2 * _NBUF)
        ),
    )
    def k(words_hbm, wt_hbm, tt_hbm, wout_hbm, tout_hbm, *rest):
        widx = rest[:_NBUF]
        tidx = rest[_NBUF:2 * _NBUF]
        wbufs = rest[2 * _NBUF:3 * _NBUF]
        tbufs = rest[3 * _NBUF:4 * _NBUF]
        wsems = rest[4 * _NBUF:5 * _NBUF]
        tsems = rest[5 * _NBUF:6 * _NBUF]
        wid = lax.axis_index("s") * 2 + lax.axis_index("c")
        base = wid * _PER_TILE
        pend = [None] * _NBUF
        for j in range(_NCHUNK):
            b = j % _NBUF
            if pend[b] is not None:
                pw, pt, pj = pend[b]
                pw.wait()
                pt.wait()
                off = base + pj * _CHUNK
                pltpu.sync_copy(wbufs[b], wout_hbm.at[pl.ds(off, _CHUNK)])
                pltpu.sync_copy(tbufs[b], tout_hbm.at[pl.ds(off, _CHUNK)])
            pltpu.sync_copy(words_hbm.at[pl.ds(base + j * _CHUNK, _CHUNK)],
                            widx[b])
            for i in range(_CHUNK // 16):
                w = widx[b][pl.ds(i * 16, 16)]
                t = jnp.minimum(jnp.maximum(w - _WORD_RANGE, 0),
                                _NUM_TRAINABLE - 1)
                tidx[b][pl.ds(i * 16, 16)] = t
            cw = pltpu.async_copy(wt_hbm.at[widx[b]], wbufs[b], wsems[b])
            ct = pltpu.async_copy(tt_hbm.at[tidx[b]], tbufs[b], tsems[b])
            pend[b] = (cw, ct, j)
        for j in range(_NCHUNK, _NCHUNK + _NBUF):
            b = j % _NBUF
            pw, pt, pj = pend[b]
            pw.wait()
            pt.wait()
            off = base + pj * _CHUNK
            pltpu.sync_copy(wbufs[b], wout_hbm.at[pl.ds(off, _CHUNK)])
            pltpu.sync_copy(tbufs[b], tout_hbm.at[pl.ds(off, _CHUNK)])

    return k(words_flat, word_table, trainable_table)


def _tc_body(wrows, trows, wcol, chars, ct, ck, cb,
             wt1, bt1, wh1, bh1, wt2, bt2, wh2, bh2, out):
    f32 = jnp.float32

    def mm(a, b):
        return lax.dot_general(a, b, (((1,), (0,)), ((), ())),
                               preferred_element_type=f32)

    mask = (wcol[...] > _WORD_RANGE).astype(f32)
    wv = wrows[...][:, :_WORD_DIM]
    tv = trows[...][:, :_WORD_DIM]
    wr = wv + jnp.maximum(tv, 0.0) * mask

    chars_blk = chars[...]
    table = ct[...]
    emb = []
    for c in range(_C):
        col = chars_blk[:, c:c + 1]
        oh = (col == lax.broadcasted_iota(jnp.int32, (_R, _CV_PAD), 1))
        emb.append(mm(oh.astype(jnp.bfloat16), table).astype(jnp.bfloat16))

    ckv = ck[...]
    pooled = None
    for c in range(_C):
        acc = None
        for k in range(_K):
            cc = c + k - 2
            if 0 <= cc < _C:
                y = mm(emb[cc], ckv[k])
                acc = y if acc is None else acc + y
        pooled = acc if pooled is None else jnp.maximum(pooled, acc)
    pooled = jnp.maximum(pooled + cb[...], 0.0)

    x = jnp.concatenate([wr, pooled], axis=1)
    for wt, bt, wh, bh in ((wt1, bt1, wh1, bh1), (wt2, bt2, wh2, bh2)):
        xb = x.astype(jnp.bfloat16)
        t = jax.nn.sigmoid(mm(xb, wt[...]) + bt[...])
        h = jnp.maximum(mm(xb, wh[...]) + bh[...], 0.0)
        x = t * h + (1.0 - t) * x
    out[...] = x


def _tc_forward(wrows, trows, words_col, chars2d, ct_bf, ck_bf, cb2,
                hw_args):
    row = pl.BlockSpec((_R, None), lambda i: (i, 0))

    def full(shape):
        return pl.BlockSpec(shape, lambda i: tuple(0 for _ in shape))

    in_specs = [
        pl.BlockSpec((_R, _WPAD), lambda i: (i, 0)),
        pl.BlockSpec((_R, _WPAD), lambda i: (i, 0)),
        pl.BlockSpec((_R, 1), lambda i: (i, 0)),
        pl.BlockSpec((_R, _C), lambda i: (i, 0)),
        full((_CV_PAD, _CHAR_DIM)),
        full((_K, _CHAR_DIM, _CHAR_DIM)),
        full((1, _CHAR_DIM)),
    ]
    for _ in range(2):
        in_specs += [full((_D, _D)), full((1, _D)),
                     full((_D, _D)), full((1, _D))]

    return pl.pallas_call(
        _tc_body,
        grid=(_NBLK,),
        in_specs=in_specs,
        out_specs=pl.BlockSpec((_R, _D), lambda i: (i, 0)),
        out_shape=jax.ShapeDtypeStruct((_BS, _D), jnp.float32),
        compiler_params=pltpu.CompilerParams(
            dimension_semantics=("arbitrary",)),
    )(wrows, trows, words_col, chars2d, ct_bf, ck_bf, cb2, *hw_args)


def kernel(words, chars, word_table, trainable_table, char_table, conv_k,
           conv_b, hw1_wt, hw1_bt, hw1_wh, hw1_bh, hw2_wt, hw2_bt, hw2_wh,
           hw2_bh):
    words_flat = words.reshape(_BS)
    wt_pad = jnp.pad(word_table, ((0, 0), (0, _WPAD - _WORD_DIM)))
    tt_pad = jnp.pad(trainable_table, ((0, 0), (0, _WPAD - _WORD_DIM)))
    wrows, trows = _sc_gather(words_flat, wt_pad, tt_pad)

    ct_bf = jnp.zeros((_CV_PAD, _CHAR_DIM), jnp.bfloat16)
    ct_bf = ct_bf.at[:_CHAR_VOCAB].set(char_table.astype(jnp.bfloat16))
    ck_bf = conv_k.astype(jnp.bfloat16)
    cb2 = conv_b.reshape(1, _CHAR_DIM)
    hw_args = [a.astype(jnp.bfloat16) if a.ndim == 2 else a.reshape(1, _D)
               for a in (hw1_wt, hw1_bt, hw1_wh, hw1_bh,
                         hw2_wt, hw2_bt, hw2_wh, hw2_bh)]

    out = _tc_forward(wrows, trows, words_flat.reshape(_BS, 1),
                      chars.reshape(_BS, _C), ct_bf, ck_bf, cb2, hw_args)
    return out.reshape(_B, _S, _D)


# speedup vs baseline: 1.7960x; 1.7960x over previous
import functools

import jax
import jax.numpy as jnp
from jax import lax
from jax.experimental import pallas as pl
from jax.experimental.pallas import tpu as pltpu
from jax.experimental.pallas import tpu_sc as plsc

_VOCAB = 100001
_NUM_TRAINABLE = 1001
_CHAR_VOCAB = 1301
_WORD_DIM = 300
_CHAR_DIM = 200
_K = 5
_B, _S, _C = 1024, 20, 16
_WORD_RANGE = _VOCAB - _NUM_TRAINABLE
_D = _WORD_DIM + _CHAR_DIM
_BS = _B * _S

_NW = 32
_PER_TILE = _BS // _NW
_CHUNK = 64
_NCHUNK = _PER_TILE // _CHUNK
_NBUF = 3
_WPAD = 384
_TR_PAD = 1008

_R = 256
_NBLK = _BS // _R
_CV_PAD = 1312


def _sc_gather(words_flat, word_table):
    mesh = plsc.VectorSubcoreMesh(core_axis_name="c", subcore_axis_name="s")

    @functools.partial(
        pl.kernel,
        out_type=jax.ShapeDtypeStruct((_BS, _WPAD), jnp.float32),
        mesh=mesh,
        scratch_types=(
            [pltpu.VMEM((_CHUNK,), jnp.int32)] * _NBUF
            + [pltpu.VMEM((_CHUNK, _WPAD), jnp.float32)] * _NBUF
            + [pltpu.SemaphoreType.DMA] * _NBUF
        ),
    )
    def k(words_hbm, wt_hbm, wout_hbm, *rest):
        widx = rest[:_NBUF]
        wbufs = rest[_NBUF:2 * _NBUF]
        wsems = rest[2 * _NBUF:3 * _NBUF]
        wid = lax.axis_index("s") * 2 + lax.axis_index("c")
        base = wid * _PER_TILE
        pend = [None] * _NBUF
        for j in range(_NCHUNK):
            b = j % _NBUF
            if pend[b] is not None:
                pw, pj = pend[b]
                pw.wait()
                off = base + pj * _CHUNK
                pltpu.sync_copy(wbufs[b], wout_hbm.at[pl.ds(off, _CHUNK)])
            pltpu.sync_copy(words_hbm.at[pl.ds(base + j * _CHUNK, _CHUNK)],
                            widx[b])
            cw = pltpu.async_copy(wt_hbm.at[widx[b]], wbufs[b], wsems[b])
            pend[b] = (cw, j)
        for j in range(_NCHUNK, _NCHUNK + _NBUF):
            b = j % _NBUF
            pw, pj = pend[b]
            pw.wait()
            off = base + pj * _CHUNK
            pltpu.sync_copy(wbufs[b], wout_hbm.at[pl.ds(off, _CHUNK)])

    return k(words_flat, word_table)


def _tc_body(wrows, wcol, chars, tt, ct, ck, cb,
             wt1, bt1, wh1, bh1, wt2, bt2, wh2, bh2, out):
    f32 = jnp.float32

    def mm(a, b):
        return lax.dot_general(a, b, (((1,), (0,)), ((), ())),
                               preferred_element_type=f32)

    wcolv = wcol[...]
    mask = (wcolv > _WORD_RANGE).astype(f32)
    tr_idx = jnp.clip(wcolv - _WORD_RANGE, 0, _NUM_TRAINABLE - 1)
    oh_tr = (tr_idx == lax.broadcasted_iota(jnp.int32, (_R, _TR_PAD), 1))
    tv = mm(oh_tr.astype(jnp.bfloat16), tt[...])
    wv = wrows[...][:, :_WORD_DIM]
    wr = wv + jnp.maximum(tv, 0.0) * mask

    chars_blk = chars[...]
    table = ct[...]
    emb = []
    for c in range(_C):
        col = chars_blk[:, c:c + 1]
        oh = (col == lax.broadcasted_iota(jnp.int32, (_R, _CV_PAD), 1))
        emb.append(mm(oh.astype(jnp.bfloat16), table).astype(jnp.bfloat16))

    ckv = ck[...]
    pooled = None
    for c in range(_C):
        acc = None
        for k in range(_K):
            cc = c + k - 2
            if 0 <= cc < _C:
                y = mm(emb[cc], ckv[k])
                acc = y if acc is None else acc + y
        pooled = acc if pooled is None else jnp.maximum(pooled, acc)
    pooled = jnp.maximum(pooled + cb[...], 0.0)

    x = jnp.concatenate([wr, pooled], axis=1)
    for wt, bt, wh, bh in ((wt1, bt1, wh1, bh1), (wt2, bt2, wh2, bh2)):
        xb = x.astype(jnp.bfloat16)
        t = jax.nn.sigmoid(mm(xb, wt[...]) + bt[...])
        h = jnp.maximum(mm(xb, wh[...]) + bh[...], 0.0)
        x = t * h + (1.0 - t) * x
    out[...] = x


def _tc_forward(wrows, words_col, chars2d, tt_bf, ct_bf, ck_bf, cb2,
                hw_args):
    row = pl.BlockSpec((_R, None), lambda i: (i, 0))

    def full(shape):
        return pl.BlockSpec(shape, lambda i: tuple(0 for _ in shape))

    in_specs = [
        pl.BlockSpec((_R, _WPAD), lambda i: (i, 0)),
        pl.BlockSpec((_R, 1), lambda i: (i, 0)),
        pl.BlockSpec((_R, _C), lambda i: (i, 0)),
        full((_TR_PAD, _WORD_DIM)),
        full((_CV_PAD, _CHAR_DIM)),
        full((_K, _CHAR_DIM, _CHAR_DIM)),
        full((1, _CHAR_DIM)),
    ]
    for _ in range(2):
        in_specs += [full((_D, _D)), full((1, _D)),
                     full((_D, _D)), full((1, _D))]

    return pl.pallas_call(
        _tc_body,
        grid=(_NBLK,),
        in_specs=in_specs,
        out_specs=pl.BlockSpec((_R, _D), lambda i: (i, 0)),
        out_shape=jax.ShapeDtypeStruct((_BS, _D), jnp.float32),
        compiler_params=pltpu.CompilerParams(
            dimension_semantics=("arbitrary",)),
    )(wrows, words_col, chars2d, tt_bf, ct_bf, ck_bf, cb2, *hw_args)


def kernel(words, chars, word_table, trainable_table, char_table, conv_k,
           conv_b, hw1_wt, hw1_bt, hw1_wh, hw1_bh, hw2_wt, hw2_bt, hw2_wh,
           hw2_bh):
    words_flat = words.reshape(_BS)
    wt_pad = jnp.pad(word_table, ((0, 0), (0, _WPAD - _WORD_DIM)))
    wrows = _sc_gather(words_flat, wt_pad)

    tt_bf = jnp.zeros((_TR_PAD, _WORD_DIM), jnp.bfloat16)
    tt_bf = tt_bf.at[:_NUM_TRAINABLE].set(trainable_table.astype(jnp.bfloat16))
    ct_bf = jnp.zeros((_CV_PAD, _CHAR_DIM), jnp.bfloat16)
    ct_bf = ct_bf.at[:_CHAR_VOCAB].set(char_table.astype(jnp.bfloat16))
    ck_bf = conv_k.astype(jnp.bfloat16)
    cb2 = conv_b.reshape(1, _CHAR_DIM)
    hw_args = [a.astype(jnp.bfloat16) if a.ndim == 2 else a.reshape(1, _D)
               for a in (hw1_wt, hw1_bt, hw1_wh, hw1_bh,
                         hw2_wt, hw2_bt, hw2_wh, hw2_bh)]

    out = _tc_forward(wrows, words_flat.reshape(_BS, 1),
                      chars.reshape(_BS, _C), tt_bf, ct_bf, ck_bf, cb2, hw_args)
    return out.reshape(_B, _S, _D)


# speedup vs baseline: 1.8554x; 1.0331x over previous
import functools

import jax
import jax.numpy as jnp
from jax import lax
from jax.experimental import pallas as pl
from jax.experimental.pallas import tpu as pltpu
from jax.experimental.pallas import tpu_sc as plsc

_VOCAB = 100001
_NUM_TRAINABLE = 1001
_CHAR_VOCAB = 1301
_WORD_DIM = 300
_CHAR_DIM = 200
_K = 5
_B, _S, _C = 1024, 20, 16
_WORD_RANGE = _VOCAB - _NUM_TRAINABLE
_D = _WORD_DIM + _CHAR_DIM
_BS = _B * _S

_NW = 32
_PER_TILE = _BS // _NW
_CHUNK = 64
_NCHUNK = _PER_TILE // _CHUNK
_NBUF = 3
_WPAD = 384
_TR_PAD = 1008

_R = 1280
_NBLK = _BS // _R
_RS = _R // _S
_CV_PAD = 1312


def _sc_gather(words_flat, word_table):
    mesh = plsc.VectorSubcoreMesh(core_axis_name="c", subcore_axis_name="s")

    @functools.partial(
        pl.kernel,
        out_type=jax.ShapeDtypeStruct((_BS, _WPAD), jnp.float32),
        mesh=mesh,
        scratch_types=(
            [pltpu.VMEM((_CHUNK,), jnp.int32)] * _NBUF
            + [pltpu.VMEM((_CHUNK, _WPAD), jnp.float32)] * _NBUF
            + [pltpu.SemaphoreType.DMA] * _NBUF
        ),
    )
    def k(words_hbm, wt_hbm, wout_hbm, *rest):
        widx = rest[:_NBUF]
        wbufs = rest[_NBUF:2 * _NBUF]
        wsems = rest[2 * _NBUF:3 * _NBUF]
        wid = lax.axis_index("s") * 2 + lax.axis_index("c")
        base = wid * _PER_TILE
        pend = [None] * _NBUF
        for j in range(_NCHUNK):
            b = j % _NBUF
            if pend[b] is not None:
                pw, pj = pend[b]
                pw.wait()
                off = base + pj * _CHUNK
                pltpu.sync_copy(wbufs[b], wout_hbm.at[pl.ds(off, _CHUNK)])
            pltpu.sync_copy(words_hbm.at[pl.ds(base + j * _CHUNK, _CHUNK)],
                            widx[b])
            cw = pltpu.async_copy(wt_hbm.at[widx[b]], wbufs[b], wsems[b])
            pend[b] = (cw, j)
        for j in range(_NCHUNK, _NCHUNK + _NBUF):
            b = j % _NBUF
            pw, pj = pend[b]
            pw.wait()
            off = base + pj * _CHUNK
            pltpu.sync_copy(wbufs[b], wout_hbm.at[pl.ds(off, _CHUNK)])

    return k(words_flat, word_table)


def _tc_body(wrows, wcol, chars, tt, ct, ck, cb,
             wt1, bt1, wh1, bh1, wt2, bt2, wh2, bh2, out):
    f32 = jnp.float32

    def mm(a, b):
        return lax.dot_general(a, b, (((1,), (0,)), ((), ())),
                               preferred_element_type=f32)

    wcolv = wcol[...]
    mask = (wcolv > _WORD_RANGE).astype(f32)
    tr_idx = jnp.clip(wcolv - _WORD_RANGE, 0, _NUM_TRAINABLE - 1)
    oh_tr = (tr_idx == lax.broadcasted_iota(jnp.int32, (_R, _TR_PAD), 1))
    tv = mm(oh_tr.astype(jnp.bfloat16), tt[...])
    wv = wrows[...][:, :_WORD_DIM]
    wr = wv + jnp.maximum(tv, 0.0) * mask

    chars_blk = chars[...]
    table = ct[...]
    emb = []
    for c in range(_C):
        col = chars_blk[:, c:c + 1]
        oh = (col == lax.broadcasted_iota(jnp.int32, (_R, _CV_PAD), 1))
        emb.append(mm(oh.astype(jnp.bfloat16), table).astype(jnp.bfloat16))

    ckv = ck[...]
    pooled = None
    for c in range(_C):
        acc = None
        for k in range(_K):
            cc = c + k - 2
            if 0 <= cc < _C:
                y = mm(emb[cc], ckv[k])
                acc = y if acc is None else acc + y
        pooled = acc if pooled is None else jnp.maximum(pooled, acc)
    pooled = jnp.maximum(pooled + cb[...], 0.0)

    x = jnp.concatenate([wr, pooled], axis=1)
    for wt, bt, wh, bh in ((wt1, bt1, wh1, bh1), (wt2, bt2, wh2, bh2)):
        xb = x.astype(jnp.bfloat16)
        t = jax.nn.sigmoid(mm(xb, wt[...]) + bt[...])
        h = jnp.maximum(mm(xb, wh[...]) + bh[...], 0.0)
        x = t * h + (1.0 - t) * x
    out[...] = x.reshape(_RS, _S, _D)


def _tc_forward(wrows, words_col, chars2d, tt_bf, ct_bf, ck_bf, cb2,
                hw_args):
    row = pl.BlockSpec((_R, None), lambda i: (i, 0))

    def full(shape):
        return pl.BlockSpec(shape, lambda i: tuple(0 for _ in shape))

    in_specs = [
        pl.BlockSpec((_R, _WPAD), lambda i: (i, 0)),
        pl.BlockSpec((_R, 1), lambda i: (i, 0)),
        pl.BlockSpec((_R, _C), lambda i: (i, 0)),
        full((_TR_PAD, _WORD_DIM)),
        full((_CV_PAD, _CHAR_DIM)),
        full((_K, _CHAR_DIM, _CHAR_DIM)),
        full((1, _CHAR_DIM)),
    ]
    for _ in range(2):
        in_specs += [full((_D, _D)), full((1, _D)),
                     full((_D, _D)), full((1, _D))]

    return pl.pallas_call(
        _tc_body,
        grid=(_NBLK,),
        in_specs=in_specs,
        out_specs=pl.BlockSpec((_RS, _S, _D), lambda i: (i, 0, 0)),
        out_shape=jax.ShapeDtypeStruct((_B, _S, _D), jnp.float32),
        compiler_params=pltpu.CompilerParams(
            dimension_semantics=("arbitrary",)),
    )(wrows, words_col, chars2d, tt_bf, ct_bf, ck_bf, cb2, *hw_args)


def kernel(words, chars, word_table, trainable_table, char_table, conv_k,
           conv_b, hw1_wt, hw1_bt, hw1_wh, hw1_bh, hw2_wt, hw2_bt, hw2_wh,
           hw2_bh):
    words_flat = words.reshape(_BS)
    wt_pad = jnp.pad(word_table, ((0, 0), (0, _WPAD - _WORD_DIM)))
    wrows = _sc_gather(words_flat, wt_pad)

    tt_bf = jnp.zeros((_TR_PAD, _WORD_DIM), jnp.bfloat16)
    tt_bf = tt_bf.at[:_NUM_TRAINABLE].set(trainable_table.astype(jnp.bfloat16))
    ct_bf = jnp.zeros((_CV_PAD, _CHAR_DIM), jnp.bfloat16)
    ct_bf = ct_bf.at[:_CHAR_VOCAB].set(char_table.astype(jnp.bfloat16))
    ck_bf = conv_k.astype(jnp.bfloat16)
    cb2 = conv_b.reshape(1, _CHAR_DIM)
    hw_args = [a.astype(jnp.bfloat16) if a.ndim == 2 else a.reshape(1, _D)
               for a in (hw1_wt, hw1_bt, hw1_wh, hw1_bh,
                         hw2_wt, hw2_bt, hw2_wh, hw2_bh)]

    return _tc_forward(wrows, words_flat.reshape(_BS, 1),
                       chars.reshape(_BS, _C), tt_bf, ct_bf, ck_bf, cb2,
                       hw_args)


# speedup vs baseline: 1.8631x; 1.0042x over previous
import functools

import jax
import jax.numpy as jnp
from jax import lax
from jax.experimental import pallas as pl
from jax.experimental.pallas import tpu as pltpu
from jax.experimental.pallas import tpu_sc as plsc

_VOCAB = 100001
_NUM_TRAINABLE = 1001
_CHAR_VOCAB = 1301
_WORD_DIM = 300
_CHAR_DIM = 200
_K = 5
_B, _S, _C = 1024, 20, 16
_WORD_RANGE = _VOCAB - _NUM_TRAINABLE
_D = _WORD_DIM + _CHAR_DIM
_BS = _B * _S

_NW = 32
_PER_TILE = _BS // _NW
_CHUNK = 64
_NCHUNK = _PER_TILE // _CHUNK
_NBUF = 3
_WPAD = 384
_TR_PAD = 1008

_R = 1280
_NBLK = _BS // _R
_RS = _R // _S
_CV_PAD = 1312


def _sc_gather(words_flat, word_table):
    mesh = plsc.VectorSubcoreMesh(core_axis_name="c", subcore_axis_name="s")

    @functools.partial(
        pl.kernel,
        out_type=jax.ShapeDtypeStruct((_BS, _WPAD), jnp.float32),
        mesh=mesh,
        scratch_types=(
            [pltpu.VMEM((_CHUNK,), jnp.int32)] * _NBUF
            + [pltpu.VMEM((_CHUNK, _WPAD), jnp.float32)] * _NBUF
            + [pltpu.SemaphoreType.DMA] * _NBUF
        ),
    )
    def k(words_hbm, wt_hbm, wout_hbm, *rest):
        widx = rest[:_NBUF]
        wbufs = rest[_NBUF:2 * _NBUF]
        wsems = rest[2 * _NBUF:3 * _NBUF]
        wid = lax.axis_index("s") * 2 + lax.axis_index("c")
        base = wid * _PER_TILE
        pend = [None] * _NBUF
        for j in range(_NCHUNK):
            b = j % _NBUF
            if pend[b] is not None:
                pw, pj = pend[b]
                pw.wait()
                off = base + pj * _CHUNK
                pltpu.sync_copy(wbufs[b], wout_hbm.at[pl.ds(off, _CHUNK)])
            pltpu.sync_copy(words_hbm.at[pl.ds(base + j * _CHUNK, _CHUNK)],
                            widx[b])
            cw = pltpu.async_copy(wt_hbm.at[widx[b]], wbufs[b], wsems[b])
            pend[b] = (cw, j)
        for j in range(_NCHUNK, _NCHUNK + _NBUF):
            b = j % _NBUF
            pw, pj = pend[b]
            pw.wait()
            off = base + pj * _CHUNK
            pltpu.sync_copy(wbufs[b], wout_hbm.at[pl.ds(off, _CHUNK)])

    return k(words_flat, word_table)


def _tc_char_body(chars, ct, ck, cb, out):
    f32 = jnp.float32

    def mm(a, b):
        return lax.dot_general(a, b, (((1,), (0,)), ((), ())),
                               preferred_element_type=f32)

    chars_blk = chars[...]
    table = ct[...]
    emb = []
    for c in range(_C):
        col = chars_blk[:, c:c + 1]
        oh = (col == lax.broadcasted_iota(jnp.int32, (_R, _CV_PAD), 1))
        emb.append(mm(oh.astype(jnp.bfloat16), table).astype(jnp.bfloat16))

    ckv = ck[...]
    pooled = None
    for c in range(_C):
        acc = None
        for k in range(_K):
            cc = c + k - 2
            if 0 <= cc < _C:
                y = mm(emb[cc], ckv[k])
                acc = y if acc is None else acc + y
        pooled = acc if pooled is None else jnp.maximum(pooled, acc)
    out[...] = jnp.maximum(pooled + cb[...], 0.0)


def _tc_mix_body(wrows, wcol, pooled, tt,
                 wt1, bt1, wh1, bh1, wt2, bt2, wh2, bh2, out):
    f32 = jnp.float32

    def mm(a, b):
        return lax.dot_general(a, b, (((1,), (0,)), ((), ())),
                               preferred_element_type=f32)

    wcolv = wcol[...]
    mask = (wcolv > _WORD_RANGE).astype(f32)
    tr_idx = jnp.clip(wcolv - _WORD_RANGE, 0, _NUM_TRAINABLE - 1)
    oh_tr = (tr_idx == lax.broadcasted_iota(jnp.int32, (_R, _TR_PAD), 1))
    tv = mm(oh_tr.astype(jnp.bfloat16), tt[...])
    wv = wrows[...][:, :_WORD_DIM]
    wr = wv + jnp.maximum(tv, 0.0) * mask

    x = jnp.concatenate([wr, pooled[...]], axis=1)
    for wt, bt, wh, bh in ((wt1, bt1, wh1, bh1), (wt2, bt2, wh2, bh2)):
        xb = x.astype(jnp.bfloat16)
        t = jax.nn.sigmoid(mm(xb, wt[...]) + bt[...])
        h = jnp.maximum(mm(xb, wh[...]) + bh[...], 0.0)
        x = t * h + (1.0 - t) * x
    out[...] = x.reshape(_RS, _S, _D)


def _tc_char(chars2d, ct_bf, ck_bf, cb2):
    def full(shape):
        return pl.BlockSpec(shape, lambda i: tuple(0 for _ in shape))

    return pl.pallas_call(
        _tc_char_body,
        grid=(_NBLK,),
        in_specs=[
            pl.BlockSpec((_R, _C), lambda i: (i, 0)),
            full((_CV_PAD, _CHAR_DIM)),
            full((_K, _CHAR_DIM, _CHAR_DIM)),
            full((1, _CHAR_DIM)),
        ],
        out_specs=pl.BlockSpec((_R, _CHAR_DIM), lambda i: (i, 0)),
        out_shape=jax.ShapeDtypeStruct((_BS, _CHAR_DIM), jnp.float32),
        compiler_params=pltpu.CompilerParams(
            dimension_semantics=("arbitrary",)),
    )(chars2d, ct_bf, ck_bf, cb2)


def _tc_mix(wrows, words_col, pooled, tt_bf, hw_args):
    def full(shape):
        return pl.BlockSpec(shape, lambda i: tuple(0 for _ in shape))

    in_specs = [
        pl.BlockSpec((_R, _WPAD), lambda i: (i, 0)),
        pl.BlockSpec((_R, 1), lambda i: (i, 0)),
        pl.BlockSpec((_R, _CHAR_DIM), lambda i: (i, 0)),
        full((_TR_PAD, _WORD_DIM)),
    ]
    for _ in range(2):
        in_specs += [full((_D, _D)), full((1, _D)),
                     full((_D, _D)), full((1, _D))]

    return pl.pallas_call(
        _tc_mix_body,
        grid=(_NBLK,),
        in_specs=in_specs,
        out_specs=pl.BlockSpec((_RS, _S, _D), lambda i: (i, 0, 0)),
        out_shape=jax.ShapeDtypeStruct((_B, _S, _D), jnp.float32),
        compiler_params=pltpu.CompilerParams(
            dimension_semantics=("arbitrary",)),
    )(wrows, words_col, pooled, tt_bf, *hw_args)


def kernel(words, chars, word_table, trainable_table, char_table, conv_k,
           conv_b, hw1_wt, hw1_bt, hw1_wh, hw1_bh, hw2_wt, hw2_bt, hw2_wh,
           hw2_bh):
    words_flat = words.reshape(_BS)
    wt_pad = jnp.pad(word_table, ((0, 0), (0, _WPAD - _WORD_DIM)))
    wrows = _sc_gather(words_flat, wt_pad)

    tt_bf = jnp.zeros((_TR_PAD, _WORD_DIM), jnp.bfloat16)
    tt_bf = tt_bf.at[:_NUM_TRAINABLE].set(trainable_table.astype(jnp.bfloat16))
    ct_bf = jnp.zeros((_CV_PAD, _CHAR_DIM), jnp.bfloat16)
    ct_bf = ct_bf.at[:_CHAR_VOCAB].set(char_table.astype(jnp.bfloat16))
    ck_bf = conv_k.astype(jnp.bfloat16)
    cb2 = conv_b.reshape(1, _CHAR_DIM)
    hw_args = [a.astype(jnp.bfloat16) if a.ndim == 2 else a.reshape(1, _D)
               for a in (hw1_wt, hw1_bt, hw1_wh, hw1_bh,
                         hw2_wt, hw2_bt, hw2_wh, hw2_bh)]

    pooled = _tc_char(chars.reshape(_BS, _C), ct_bf, ck_bf, cb2)
    return _tc_mix(wrows, words_flat.reshape(_BS, 1), pooled, tt_bf, hw_args)
